# baseline (speedup 1.0000x reference)
import functools
import math

import jax
import jax.numpy as jnp
from jax import lax
from jax.experimental import pallas as pl
from jax.experimental.pallas import tpu as pltpu
from jax.experimental.pallas import tpu_sc as plsc

_B, _N, _D = 32, 1000, 1024
_QL = 20
_TOPK = 40
_BOXD = 4
_L = 16
_NCHUNK = (_N + _L - 1) // _L
_NPAD = _NCHUNK * _L
_NEG = float("-inf")
_INV_SQRT_D = 1.0 / math.sqrt(_D)


_NCAND = 64


def _query_kernel(q_ref, qm_ref, wqb_ref, bq_ref, wsa_ref, wocrb_ref,
                  bocr_ref, gq_ref, u_ref, qb_ref):
    xb = q_ref[0].astype(jnp.bfloat16)
    qm = qm_ref[0]
    qp = lax.dot_general(xb, wqb_ref[...], (((1,), (1,)), ((), ())),
                         preferred_element_type=jnp.float32)
    qp = qp + bq_ref[...]
    qpb = qp.astype(jnp.bfloat16)
    logits = lax.dot_general(wsa_ref[...], qpb, (((1,), (1,)), ((), ())),
                             preferred_element_type=jnp.float32)
    m = jnp.max(logits, axis=1, keepdims=True)
    e = jnp.exp(logits - m)
    attn = e / jnp.sum(e, axis=1, keepdims=True)
    attn = attn * qm
    attn = attn / (jnp.sum(attn, axis=1, keepdims=True) + 1e-12)
    gq = lax.dot_general(attn.astype(jnp.bfloat16), qpb,
                         (((1,), (0,)), ((), ())),
                         preferred_element_type=jnp.float32)
    gqb = gq.astype(jnp.bfloat16)
    gq_ref[0] = gqb
    u = lax.dot_general(gqb, wocrb_ref[...], (((1,), (0,)), ((), ())),
                        preferred_element_type=jnp.float32)
    u_ref[0] = u.astype(jnp.bfloat16)
    qb = jnp.sum(gq * bocr_ref[...], axis=1, keepdims=True)
    qb_ref[0] = jnp.broadcast_to(qb, (1, 128))


def _score_kernel(ocr_ref, mask_ref, u_ref, qb_ref, out_ref):
    xb = ocr_ref[0].astype(jnp.bfloat16)
    s = lax.dot_general(u_ref[0], xb, (((1,), (1,)), ((), ())),
                        preferred_element_type=jnp.float32)
    s = (s + qb_ref[0][:, :1]) * _INV_SQRT_D
    out_ref[0] = jnp.where(mask_ref[0] > 0, s, jnp.float32(-1e4))


def _preselect_kernel(s_ref, cand_ref, oh_ref):
    work = s_ref[:, 0, :]
    niota = lax.broadcasted_iota(jnp.int32, (_B, _N), 1)
    ties = []
    for r in range(_NCAND):
        mv = jnp.max(work, axis=1, keepdims=True)
        tie = jnp.min(jnp.where(work == mv, niota, jnp.int32(_N)),
                      axis=1, keepdims=True)
        hit = niota == tie
        oh_ref[:, r, :] = hit.astype(jnp.bfloat16)
        work = jnp.where(hit, jnp.float32(-jnp.inf), work)
        ties.append(tie)
    cand_ref[:, 0, :] = jnp.concatenate(ties, axis=1)


def _rescore_kernel(cand_ref, ocr_ref, oh_ref, mask_ref, gq_ref, wocrb_ref,
                    bocr_ref, m_ref, feats_ref, sem):
    b = pl.program_id(0)
    for c in range(_NCAND):
        pltpu.make_async_copy(
            ocr_ref.at[b, cand_ref[b, 0, c]], feats_ref.at[c], sem,
        ).start()
    for c in range(_NCAND):
        pltpu.make_async_copy(
            ocr_ref.at[b, cand_ref[b, 0, c]], feats_ref.at[c], sem,
        ).wait()
    kp = lax.dot_general(feats_ref[...].astype(jnp.bfloat16), wocrb_ref[...],
                         (((1,), (1,)), ((), ())),
                         preferred_element_type=jnp.float32)
    kpb = (kp + bocr_ref[...]).astype(jnp.bfloat16)
    m64 = lax.dot_general(gq_ref[0], kpb, (((1,), (1,)), ((), ())),
                          preferred_element_type=jnp.float32)
    cmask = lax.dot_general(mask_ref[0].astype(jnp.bfloat16), oh_ref[0],
                            (((1,), (1,)), ((), ())),
                            preferred_element_type=jnp.float32)
    m_ref[0] = jnp.where(cmask > 0, m64 * _INV_SQRT_D, jnp.float32(-1e4))


def _sc_topk_gather(cand_hbm, m_hbm, mask_hbm, box_hbm, out_hbm,
                    cbuf, vbuf, fbuf, mbuf, boxbuf, selbuf, obuf):
    nc = 2
    b = lax.axis_index("s") * nc + lax.axis_index("c")
    lane = lax.iota(jnp.int32, _L)
    neg = jnp.full((_L,), _NEG, jnp.float32)
    big = jnp.full((_L,), 2**30, jnp.int32)
    one = jnp.full((_L,), 1.0, jnp.float32)

    pltpu.sync_copy(cand_hbm.at[pl.ds(b * _NCAND, _NCAND)], cbuf)
    pltpu.sync_copy(m_hbm.at[pl.ds(b * _NCAND, _NCAND)], vbuf)
    pltpu.sync_copy(mask_hbm.at[pl.ds(b * _N, _N)], mbuf)
    pltpu.sync_copy(box_hbm.at[pl.ds(b * _N * _BOXD, _N * _BOXD)], boxbuf)

    for c in range(_NCHUNK):
        fbuf[pl.ds(c * _L, _L)] = jnp.zeros((_L,), jnp.float32)

    def sel_round(_, carry):
        bestv, besto, bestp = neg, big, jnp.zeros((_L,), jnp.int32)
        for c in range(_NCAND // _L):
            v = vbuf[pl.ds(c * _L, _L)]
            o = cbuf[pl.ds(c * _L, _L)]
            p = c * _L + lane
            better = (v > bestv) | ((v == bestv) & (o < besto))
            bestv = jnp.where(better, v, bestv)
            besto = jnp.where(better, o, besto)
            bestp = jnp.where(better, p, bestp)
        mval = jnp.max(bestv)
        morig = jnp.min(jnp.where(bestv == mval, besto, big))
        mpos = jnp.min(jnp.where((bestv == mval) & (besto == morig),
                                 bestp, big))
        plsc.store_scatter(vbuf, [lane * 0 + mpos], neg, mask=lane == 0)
        plsc.store_scatter(fbuf, [lane * 0 + morig], one, mask=lane == 0)
        return carry

    lax.fori_loop(0, _TOPK, sel_round, jnp.int32(0))

    for c in range(3):
        selbuf[pl.ds(c * _L, _L)] = jnp.zeros((_L,), jnp.int32)

    def compact_chunk(c, cnt):
        idxv = c * _L + lane
        v = plsc.load_gather(fbuf, [idxv])
        m = v > 0.5
        pos = cnt + plsc.cumsum(m.astype(jnp.int32)) - 1
        plsc.store_scatter(selbuf, [pos], idxv, mask=m)
        return cnt + plsc.all_reduce_population_count(m)

    lax.fori_loop(0, _NCHUNK, compact_chunk, jnp.zeros((_L,), jnp.int32))

    for c in range(_TOPK * _BOXD // _L):
        p = c * _L + lane
        r = p // _BOXD
        comp = p - r * _BOXD
        si = plsc.load_gather(selbuf, [r])
        mv = plsc.load_gather(mbuf, [si])
        bv = plsc.load_gather(boxbuf, [si * _BOXD + comp])
        obuf[pl.ds(c * _L, _L)] = bv * mv
    pltpu.sync_copy(obuf, out_hbm.at[pl.ds(b * _TOPK * _BOXD,
                                           _TOPK * _BOXD)])


@functools.lru_cache(maxsize=None)
def _make_topk_gather():
    mesh = plsc.VectorSubcoreMesh(core_axis_name="c", subcore_axis_name="s")
    return pl.kernel(
        _sc_topk_gather,
        mesh=mesh,
        compiler_params=pltpu.CompilerParams(needs_layout_passes=False),
        out_type=jax.ShapeDtypeStruct((_B * _TOPK * _BOXD,), jnp.float32),
        scratch_types=[
            pltpu.VMEM((_NCAND,), jnp.int32),
            pltpu.VMEM((_NCAND,), jnp.float32),
            pltpu.VMEM((_NPAD,), jnp.float32),
            pltpu.VMEM((_N,), jnp.float32),
            pltpu.VMEM((_N * _BOXD,), jnp.float32),
            pltpu.VMEM((48,), jnp.int32),
            pltpu.VMEM((_TOPK * _BOXD,), jnp.float32),
        ],
    )


def kernel(ocr_feat, ocr_mask, frame_feat, frame_mask, q_feat, q_mask,
           ocr_box, Wq, bq, W_sa, b_sa, W_ocr, b_ocr):
    del frame_feat, frame_mask, b_sa

    wocrb = W_ocr.astype(jnp.bfloat16)
    bocr2 = b_ocr.reshape(1, _D)
    gqb, ub, qb = pl.pallas_call(
        _query_kernel,
        grid=(_B,),
        in_specs=[
            pl.BlockSpec((1, _QL, _D), lambda b: (b, 0, 0)),
            pl.BlockSpec((1, 1, _QL), lambda b: (b, 0, 0)),
            pl.BlockSpec((_D, _D), lambda b: (0, 0)),
            pl.BlockSpec((1, _D), lambda b: (0, 0)),
            pl.BlockSpec((1, _D), lambda b: (0, 0)),
            pl.BlockSpec((_D, _D), lambda b: (0, 0)),
            pl.BlockSpec((1, _D), lambda b: (0, 0)),
        ],
        out_specs=[
            pl.BlockSpec((1, 1, _D), lambda b: (b, 0, 0)),
            pl.BlockSpec((1, 1, _D), lambda b: (b, 0, 0)),
            pl.BlockSpec((1, 1, 128), lambda b: (b, 0, 0)),
        ],
        out_shape=[
            jax.ShapeDtypeStruct((_B, 1, _D), jnp.bfloat16),
            jax.ShapeDtypeStruct((_B, 1, _D), jnp.bfloat16),
            jax.ShapeDtypeStruct((_B, 1, 128), jnp.float32),
        ],
    )(q_feat, q_mask.reshape(_B, 1, _QL),
      Wq.astype(jnp.bfloat16), bq.reshape(1, _D),
      W_sa.astype(jnp.bfloat16), wocrb, bocr2)

    mask3 = ocr_mask.reshape(_B, 1, _N)
    scores3 = pl.pallas_call(
        _score_kernel,
        grid=(_B,),
        in_specs=[
            pl.BlockSpec((1, _N, _D), lambda b: (b, 0, 0)),
            pl.BlockSpec((1, 1, _N), lambda b: (b, 0, 0)),
            pl.BlockSpec((1, 1, _D), lambda b: (b, 0, 0)),
            pl.BlockSpec((1, 1, 128), lambda b: (b, 0, 0)),
        ],
        out_specs=pl.BlockSpec((1, 1, _N), lambda b: (b, 0, 0)),
        out_shape=jax.ShapeDtypeStruct((_B, 1, _N), jnp.float32),
    )(ocr_feat, mask3, ub, qb)
    scores = scores3.reshape(_B, _N)

    cand, onehot = pl.pallas_call(
        _preselect_kernel,
        out_shape=[
            jax.ShapeDtypeStruct((_B, 1, _NCAND), jnp.int32),
            jax.ShapeDtypeStruct((_B, _NCAND, _N), jnp.bfloat16),
        ],
    )(scores3)

    m64 = pl.pallas_call(
        _rescore_kernel,
        grid=(_B,),
        in_specs=[
            pl.BlockSpec(memory_space=pltpu.SMEM),
            pl.BlockSpec(memory_space=pl.ANY),
            pl.BlockSpec((1, _NCAND, _N), lambda b: (b, 0, 0)),
            pl.BlockSpec((1, 1, _N), lambda b: (b, 0, 0)),
            pl.BlockSpec((1, 1, _D), lambda b: (b, 0, 0)),
            pl.BlockSpec((_D, _D), lambda b: (0, 0)),
            pl.BlockSpec((1, _D), lambda b: (0, 0)),
        ],
        out_specs=pl.BlockSpec((1, 1, _NCAND), lambda b: (b, 0, 0)),
        out_shape=jax.ShapeDtypeStruct((_B, 1, _NCAND), jnp.float32),
        scratch_shapes=[
            pltpu.VMEM((_NCAND, _D), jnp.float32),
            pltpu.SemaphoreType.DMA,
        ],
    )(cand, ocr_feat, onehot, mask3, gqb, wocrb, bocr2)

    out_flat = _make_topk_gather()(cand.reshape(-1), m64.reshape(-1),
                                   ocr_mask.reshape(-1),
                                   ocr_box.reshape(-1))
    return out_flat.reshape(_B, _TOPK, _BOXD), scores


# speedup vs baseline: 1.1961x; 1.1961x over previous
import functools
import math

import jax
import jax.numpy as jnp
from jax import lax
from jax.experimental import pallas as pl
from jax.experimental.pallas import tpu as pltpu
from jax.experimental.pallas import tpu_sc as plsc

_B, _N, _D = 32, 1000, 1024
_QL = 20
_TOPK = 40
_BOXD = 4
_L = 16
_NCHUNK = (_N + _L - 1) // _L
_NPAD = _NCHUNK * _L
_NEG = float("-inf")
_INV_SQRT_D = 1.0 / math.sqrt(_D)


def _query_kernel(q_ref, qm_ref, wqb_ref, bq_ref, wsa_ref, gq_ref):
    xb = q_ref[0].astype(jnp.bfloat16)
    qm = qm_ref[0]
    qp = lax.dot_general(xb, wqb_ref[...], (((1,), (1,)), ((), ())),
                         preferred_element_type=jnp.float32)
    qp = qp + bq_ref[...]
    qpb = qp.astype(jnp.bfloat16)
    logits = lax.dot_general(wsa_ref[...], qpb, (((1,), (1,)), ((), ())),
                             preferred_element_type=jnp.float32)
    m = jnp.max(logits, axis=1, keepdims=True)
    e = jnp.exp(logits - m)
    attn = e / jnp.sum(e, axis=1, keepdims=True)
    attn = attn * qm
    attn = attn / (jnp.sum(attn, axis=1, keepdims=True) + 1e-12)
    gq = lax.dot_general(attn.astype(jnp.bfloat16), qpb,
                         (((1,), (0,)), ((), ())),
                         preferred_element_type=jnp.float32)
    gq_ref[0] = gq.astype(jnp.bfloat16)


def _score_kernel(ocr_ref, mask_ref, gq_ref, wocrb_ref, bocr_ref, out_ref):
    xb = ocr_ref[0].astype(jnp.bfloat16)
    kp = lax.dot_general(xb, wocrb_ref[...], (((1,), (1,)), ((), ())),
                         preferred_element_type=jnp.float32)
    kpb = (kp + bocr_ref[...]).astype(jnp.bfloat16)
    s = lax.dot_general(gq_ref[0], kpb, (((1,), (1,)), ((), ())),
                        preferred_element_type=jnp.float32)
    s = s * _INV_SQRT_D
    out_ref[0] = jnp.where(mask_ref[0] > 0, s, jnp.float32(-1e4))


def _sc_topk_gather(scores_hbm, mask_hbm, box_hbm, out_hbm,
                    sbuf, mbuf, boxbuf, selbuf, obuf):
    nc = 2
    b = lax.axis_index("s") * nc + lax.axis_index("c")
    lane = lax.iota(jnp.int32, _L)
    neg = jnp.full((_L,), _NEG, jnp.float32)

    pltpu.sync_copy(scores_hbm.at[pl.ds(b * _N, _N)], sbuf.at[pl.ds(0, _N)])
    pltpu.sync_copy(mask_hbm.at[pl.ds(b * _N, _N)], mbuf)
    pltpu.sync_copy(box_hbm.at[pl.ds(b * _N * _BOXD, _N * _BOXD)], boxbuf)

    ntail = _N - (_NCHUNK - 1) * _L
    tail = sbuf[pl.ds((_NCHUNK - 1) * _L, _L)]
    sbuf[pl.ds((_NCHUNK - 1) * _L, _L)] = jnp.where(lane < ntail, tail, neg)

    def sel_round(_, carry):
        def scan_chunk(c, st):
            bestv, besti = st
            idxv = c * _L + lane
            v = plsc.load_gather(sbuf, [idxv])
            better = v > bestv
            return (jnp.where(better, v, bestv),
                    jnp.where(better, idxv, besti))
        bestv, besti = lax.fori_loop(
            0, _NCHUNK, scan_chunk,
            (neg, jnp.zeros((_L,), jnp.int32)))
        mval = jnp.max(bestv)
        cand = jnp.where(bestv == mval, besti, jnp.int32(2**30))
        midx = jnp.min(cand)
        plsc.store_scatter(sbuf, [lane * 0 + midx], neg, mask=lane == 0)
        return carry

    lax.fori_loop(0, _TOPK, sel_round, jnp.int32(0))

    for c in range(3):
        selbuf[pl.ds(c * _L, _L)] = jnp.zeros((_L,), jnp.int32)

    def compact_chunk(c, cnt):
        idxv = c * _L + lane
        v = plsc.load_gather(sbuf, [idxv])
        m = (v == _NEG) & (idxv < _N)
        pos = cnt + plsc.cumsum(m.astype(jnp.int32)) - 1
        plsc.store_scatter(selbuf, [pos], idxv, mask=m)
        return cnt + plsc.all_reduce_population_count(m)

    lax.fori_loop(0, _NCHUNK, compact_chunk, jnp.zeros((_L,), jnp.int32))

    for c in range(_TOPK * _BOXD // _L):
        p = c * _L + lane
        r = p // _BOXD
        comp = p - r * _BOXD
        si = plsc.load_gather(selbuf, [r])
        mv = plsc.load_gather(mbuf, [si])
        bv = plsc.load_gather(boxbuf, [si * _BOXD + comp])
        obuf[pl.ds(c * _L, _L)] = bv * mv
    pltpu.sync_copy(obuf, out_hbm.at[pl.ds(b * _TOPK * _BOXD,
                                           _TOPK * _BOXD)])


@functools.lru_cache(maxsize=None)
def _make_topk_gather():
    mesh = plsc.VectorSubcoreMesh(core_axis_name="c", subcore_axis_name="s")
    return pl.kernel(
        _sc_topk_gather,
        mesh=mesh,
        compiler_params=pltpu.CompilerParams(needs_layout_passes=False),
        out_type=jax.ShapeDtypeStruct((_B * _TOPK * _BOXD,), jnp.float32),
        scratch_types=[
            pltpu.VMEM((_NPAD,), jnp.float32),
            pltpu.VMEM((_N,), jnp.float32),
            pltpu.VMEM((_N * _BOXD,), jnp.float32),
            pltpu.VMEM((48,), jnp.int32),
            pltpu.VMEM((_TOPK * _BOXD,), jnp.float32),
        ],
    )


def kernel(ocr_feat, ocr_mask, frame_feat, frame_mask, q_feat, q_mask,
           ocr_box, Wq, bq, W_sa, b_sa, W_ocr, b_ocr):
    del frame_feat, frame_mask, b_sa

    gqb = pl.pallas_call(
        _query_kernel,
        grid=(_B,),
        in_specs=[
            pl.BlockSpec((1, _QL, _D), lambda b: (b, 0, 0)),
            pl.BlockSpec((1, 1, _QL), lambda b: (b, 0, 0)),
            pl.BlockSpec((_D, _D), lambda b: (0, 0)),
            pl.BlockSpec((1, _D), lambda b: (0, 0)),
            pl.BlockSpec((1, _D), lambda b: (0, 0)),
        ],
        out_specs=pl.BlockSpec((1, 1, _D), lambda b: (b, 0, 0)),
        out_shape=jax.ShapeDtypeStruct((_B, 1, _D), jnp.bfloat16),
    )(q_feat, q_mask.reshape(_B, 1, _QL),
      Wq.astype(jnp.bfloat16), bq.reshape(1, _D),
      W_sa.astype(jnp.bfloat16))

    scores = pl.pallas_call(
        _score_kernel,
        grid=(_B,),
        in_specs=[
            pl.BlockSpec((1, _N, _D), lambda b: (b, 0, 0)),
            pl.BlockSpec((1, 1, _N), lambda b: (b, 0, 0)),
            pl.BlockSpec((1, 1, _D), lambda b: (b, 0, 0)),
            pl.BlockSpec((_D, _D), lambda b: (0, 0)),
            pl.BlockSpec((1, _D), lambda b: (0, 0)),
        ],
        out_specs=pl.BlockSpec((1, 1, _N), lambda b: (b, 0, 0)),
        out_shape=jax.ShapeDtypeStruct((_B, 1, _N), jnp.float32),
    )(ocr_feat, ocr_mask.reshape(_B, 1, _N), gqb,
      W_ocr.astype(jnp.bfloat16), b_ocr.reshape(1, _D))
    scores = scores.reshape(_B, _N)

    out_flat = _make_topk_gather()(scores.reshape(-1), ocr_mask.reshape(-1),
                                   ocr_box.reshape(-1))
    return out_flat.reshape(_B, _TOPK, _BOXD), scores
